# baseline (speedup 1.0000x reference)
import jax
import jax.numpy as jnp
from jax import lax
from jax.experimental import pallas as pl
from jax.experimental.pallas import tpu as pltpu
from jax.experimental.pallas import tpu_sc as plsc

_P, _N, _E, _D = 2, 10000, 160000, 128
_L = 3
_NEXP, _NATT = 8, 64
_NS = 16
_NC = 2
_CH = 80
_EPT = _E // _NS
_NCHUNK = _EPT // _CH
_G = 4
_IB = 6
_LCM = 12
_NLOOP = _NCHUNK // _LCM
_TAIL = _NCHUNK - _NLOOP * _LCM
_RPT = _N // _NS


def _seg_body(x_hbm, ei_hbm, out_hbm,
              sidx, didx, rows, i0, i1, i2, i3, i4, i5, g0, g1, g2, g3, acc):
    isems = (i0, i1, i2, i3, i4, i5)
    gsems = (g0, g1, g2, g3)
    c = lax.axis_index("c")
    s = lax.axis_index("s")
    ebase = s * _EPT
    r0 = s * _RPT

    def idx_issue(k, slot):
        pltpu.async_copy(ei_hbm.at[c, 0, pl.ds(ebase + k * _CH, _CH)],
                         sidx.at[slot], isems[slot])
        pltpu.async_copy(ei_hbm.at[c, 1, pl.ds(ebase + k * _CH, _CH)],
                         didx.at[slot], isems[slot])

    def gather_issue(slot, rslot):
        pltpu.make_async_copy(ei_hbm.at[c, 0, pl.ds(0, _CH)],
                              sidx.at[slot], isems[slot]).wait()
        pltpu.make_async_copy(ei_hbm.at[c, 0, pl.ds(0, _CH)],
                              didx.at[slot], isems[slot]).wait()
        pltpu.async_copy(x_hbm.at[c].at[sidx.at[slot]], rows.at[rslot],
                         gsems[rslot])

    def turn(i, b):
        ib = b % _IB
        rb = b % _G
        pltpu.make_async_copy(x_hbm.at[c].at[sidx.at[ib]], rows.at[rb],
                              gsems[rb]).wait()
        pltpu.sync_copy(rows.at[rb], acc.at[didx.at[ib]], add=True)

        @pl.when(i + _IB < _NCHUNK)
        def _():
            idx_issue(i + _IB, ib)

        @pl.when(i + _G < _NCHUNK)
        def _():
            gather_issue((b + _G) % _IB, rb)

    for k in range(_IB):
        idx_issue(k, k)
    pltpu.sync_copy(x_hbm.at[c, pl.ds(r0, _RPT)], acc.at[pl.ds(r0, _RPT)])
    plsc.subcore_barrier()
    for g in range(_G):
        gather_issue(g, g)

    def step(j, carry):
        base = j * _LCM
        for b in range(_LCM):
            turn(base + b, b)
        return carry

    lax.fori_loop(0, _NLOOP, step, 0)
    for k in range(_TAIL):
        turn(_NLOOP * _LCM + k, k)

    plsc.subcore_barrier()
    pltpu.sync_copy(acc.at[pl.ds(r0, _RPT)], out_hbm.at[c, pl.ds(r0, _RPT)])


_seg_call = None


def _seg(x3, ei):
    global _seg_call
    if _seg_call is None:
        _seg_call = pl.kernel(
            _seg_body,
            out_type=jax.ShapeDtypeStruct((_P, _N, _D), jnp.float32),
            mesh=plsc.VectorSubcoreMesh(core_axis_name="c",
                                        subcore_axis_name="s",
                                        num_cores=_NC, num_subcores=_NS),
            compiler_params=pltpu.CompilerParams(use_tc_tiling_on_sc=False),
            scratch_types=(
                [pltpu.VMEM((_IB, _CH), jnp.int32),
                 pltpu.VMEM((_IB, _CH), jnp.int32),
                 pltpu.VMEM((_G, _CH, _D), jnp.float32)]
                + [pltpu.SemaphoreType.DMA] * (_IB + _G)
                + [pltpu.VMEM_SHARED((_N, _D), jnp.float32)]
            ),
        )
    return _seg_call(x3, ei)


_ROWB = 1000
_NBLK = (_P * _N) // _ROWB
_BPG = _N // _ROWB


def _ln(t, g, b):
    mu = jnp.mean(t, axis=-1, keepdims=True)
    var = jnp.mean((t - mu) * (t - mu), axis=-1, keepdims=True)
    return (t - mu) * lax.rsqrt(var + 1e-5) * g + b


def _dot_hp(a, w):
    return jnp.dot(a, w, preferred_element_type=jnp.float32)


def _mlp_mid_body(hp_ref, idn_ref, wa_ref, ba_ref, wb_ref, bb_ref,
                  g_ref, b_ref, o_ref):
    h1 = jnp.maximum(_dot_hp(hp_ref[0], wa_ref[...]) + ba_ref[...], 0.0)
    h2 = _dot_hp(h1, wb_ref[...]) + bb_ref[...] + idn_ref[0]
    o_ref[0] = _ln(h2, g_ref[...], b_ref[...])


def _mlp_last_body(hp_ref, wa_ref, ba_ref, wb_ref, bb_ref,
                   g_ref, b_ref, w1t_ref, w2t_ref, o_ref, g8_ref):
    i = pl.program_id(0)
    h1 = jnp.maximum(_dot_hp(hp_ref[0], wa_ref[...]) + ba_ref[...], 0.0)
    h2 = _dot_hp(h1, wb_ref[...]) + bb_ref[...]
    x = _ln(h2, g_ref[...], b_ref[...])

    @pl.when(i == 0)
    def _():
        g8_ref[...] = jnp.zeros_like(g8_ref)

    p = i // _BPG
    rowsel = lax.broadcasted_iota(jnp.int32, (8, _D), 0) == p
    g8_ref[...] += jnp.where(rowsel, jnp.sum(x, axis=0, keepdims=True), 0.0)

    @pl.when(i == _NBLK - 1)
    def _():
        g8 = g8_ref[...]
        sup = jnp.tanh(jnp.dot(g8, w1t_ref[...],
                               preferred_element_type=jnp.float32))
        logits = jnp.dot(sup, w2t_ref[...],
                         preferred_element_type=jnp.float32)
        row = lax.broadcasted_iota(jnp.int32, (8, _NEXP), 0)
        logits = jnp.where(row < _P, logits, -1e30)
        m = jnp.max(logits, axis=0, keepdims=True)
        e = jnp.exp(logits - m)
        att = e / jnp.sum(e, axis=0, keepdims=True)
        o_ref[...] = lax.dot_general(att, g8, (((0,), (0,)), ((), ())),
                                     preferred_element_type=jnp.float32)


_w_spec = pl.BlockSpec((_D, _D), lambda i: (0, 0))
_v_spec = pl.BlockSpec((1, _D), lambda i: (0, 0))
_row3_spec = pl.BlockSpec((1, _ROWB, _D), lambda i: (i // _BPG, i % _BPG, 0))

_mlp_mid = pl.pallas_call(
    _mlp_mid_body,
    grid=(_NBLK,),
    in_specs=[_row3_spec, _row3_spec, _w_spec, _v_spec, _w_spec, _v_spec,
              _v_spec, _v_spec],
    out_specs=_row3_spec,
    out_shape=jax.ShapeDtypeStruct((_P, _N, _D), jnp.float32),
)

_mlp_last = pl.pallas_call(
    _mlp_last_body,
    grid=(_NBLK,),
    in_specs=[_row3_spec, _w_spec, _v_spec, _w_spec, _v_spec, _v_spec,
              _v_spec,
              pl.BlockSpec((_D, _NATT), lambda i: (0, 0)),
              pl.BlockSpec((_NATT, _NEXP), lambda i: (0, 0))],
    out_specs=pl.BlockSpec((_NEXP, _D), lambda i: (0, 0)),
    out_shape=jax.ShapeDtypeStruct((_NEXP, _D), jnp.float32),
    scratch_shapes=[pltpu.VMEM((8, _D), jnp.float32)],
)


def kernel(decomp_x, decomp_edge_index, decomp_edge_attr,
           Wa, ba, Wb, bb, lng, lnb, attw1, attw2):
    del decomp_edge_attr
    x3 = decomp_x

    out8 = None
    for l in range(_L):
        hp = _seg(x3, decomp_edge_index)
        ba_l, bb_l = ba[l].reshape(1, _D), bb[l].reshape(1, _D)
        g_l, b_l = lng[l].reshape(1, _D), lnb[l].reshape(1, _D)
        if l < _L - 1:
            x3 = _mlp_mid(hp, x3, Wa[l], ba_l, Wb[l], bb_l, g_l, b_l)
        else:
            out8 = _mlp_last(hp, Wa[l], ba_l, Wb[l], bb_l, g_l, b_l,
                             attw1.T, attw2.T)

    return out8.reshape(1, _NEXP * _D)


# speedup vs baseline: 1.0876x; 1.0876x over previous
import jax
import jax.numpy as jnp
from jax import lax
from jax.experimental import pallas as pl
from jax.experimental.pallas import tpu as pltpu
from jax.experimental.pallas import tpu_sc as plsc

_P, _N, _E, _D = 2, 10000, 160000, 128
_L = 3
_NEXP, _NATT = 8, 64
_NS = 16
_NC = 2
_CH = 80
_EPT = _E // _NS
_NCHUNK = _EPT // _CH
_G = 4
_IB = 6
_LCM = 12
_NLOOP = _NCHUNK // _LCM
_TAIL = _NCHUNK - _NLOOP * _LCM
_RPT = _N // _NS


def _seg_body(x_hbm, ei_hbm, out_hbm,
              sidx, didx, rows, i0, i1, i2, i3, i4, i5, g0, g1, g2, g3, acc):
    isems = (i0, i1, i2, i3, i4, i5)
    gsems = (g0, g1, g2, g3)
    c = lax.axis_index("c")
    s = lax.axis_index("s")
    ebase = s * _EPT
    r0 = s * _RPT

    def idx_issue(k, slot):
        pltpu.async_copy(ei_hbm.at[c, 0, pl.ds(ebase + k * _CH, _CH)],
                         sidx.at[slot], isems[slot])
        pltpu.async_copy(ei_hbm.at[c, 1, pl.ds(ebase + k * _CH, _CH)],
                         didx.at[slot], isems[slot])

    def gather_issue(slot, rslot):
        pltpu.make_async_copy(ei_hbm.at[c, 0, pl.ds(0, _CH)],
                              sidx.at[slot], isems[slot]).wait()
        pltpu.make_async_copy(ei_hbm.at[c, 0, pl.ds(0, _CH)],
                              didx.at[slot], isems[slot]).wait()
        pltpu.async_copy(x_hbm.at[c].at[sidx.at[slot]], rows.at[rslot],
                         gsems[rslot])

    def turn(i, b):
        ib = b % _IB
        rb = b % _G
        pltpu.make_async_copy(x_hbm.at[c].at[sidx.at[ib]], rows.at[rb],
                              gsems[rb]).wait()
        pltpu.sync_copy(rows.at[rb], acc.at[didx.at[ib]], add=True)

        @pl.when(i + _IB < _NCHUNK)
        def _():
            idx_issue(i + _IB, ib)

        @pl.when(i + _G < _NCHUNK)
        def _():
            gather_issue((b + _G) % _IB, rb)

    for k in range(_IB):
        idx_issue(k, k)
    pltpu.sync_copy(x_hbm.at[c, pl.ds(r0, _RPT)], acc.at[pl.ds(r0, _RPT)])
    plsc.subcore_barrier()
    for g in range(_G):
        gather_issue(g, g)

    def step(j, carry):
        base = j * _LCM
        for b in range(_LCM):
            turn(base + b, b)
        return carry

    lax.fori_loop(0, _NLOOP, step, 0)
    for k in range(_TAIL):
        turn(_NLOOP * _LCM + k, k)

    plsc.subcore_barrier()
    pltpu.sync_copy(acc.at[pl.ds(r0, _RPT)], out_hbm.at[c, pl.ds(r0, _RPT)])


_seg_call = None


def _seg(x3, ei):
    global _seg_call
    if _seg_call is None:
        _seg_call = pl.kernel(
            _seg_body,
            out_type=jax.ShapeDtypeStruct((_P, _N, _D), jnp.float32),
            mesh=plsc.VectorSubcoreMesh(core_axis_name="c",
                                        subcore_axis_name="s",
                                        num_cores=_NC, num_subcores=_NS),
            compiler_params=pltpu.CompilerParams(use_tc_tiling_on_sc=False),
            scratch_types=(
                [pltpu.VMEM((_IB, _CH), jnp.int32),
                 pltpu.VMEM((_IB, _CH), jnp.int32),
                 pltpu.VMEM((_G, _CH, _D), jnp.float32)]
                + [pltpu.SemaphoreType.DMA] * (_IB + _G)
                + [pltpu.VMEM_SHARED((_N, _D), jnp.float32)]
            ),
        )
    return _seg_call(x3, ei)


_ROWB = 5000
_NBLK = (_P * _N) // _ROWB
_BPG = _N // _ROWB


def _ln(t, g, b):
    mu = jnp.mean(t, axis=-1, keepdims=True)
    var = jnp.mean((t - mu) * (t - mu), axis=-1, keepdims=True)
    return (t - mu) * lax.rsqrt(var + 1e-5) * g + b


def _dot_hp(a, w):
    return jnp.dot(a, w, preferred_element_type=jnp.float32)


def _mlp_mid_body(hp_ref, idn_ref, wa_ref, ba_ref, wb_ref, bb_ref,
                  g_ref, b_ref, o_ref):
    h1 = jnp.maximum(_dot_hp(hp_ref[0], wa_ref[...]) + ba_ref[...], 0.0)
    h2 = _dot_hp(h1, wb_ref[...]) + bb_ref[...] + idn_ref[0]
    o_ref[0] = _ln(h2, g_ref[...], b_ref[...])


def _mlp_last_body(hp_ref, wa_ref, ba_ref, wb_ref, bb_ref,
                   g_ref, b_ref, w1t_ref, w2t_ref, o_ref, g8_ref):
    i = pl.program_id(0)
    h1 = jnp.maximum(_dot_hp(hp_ref[0], wa_ref[...]) + ba_ref[...], 0.0)
    h2 = _dot_hp(h1, wb_ref[...]) + bb_ref[...]
    x = _ln(h2, g_ref[...], b_ref[...])

    @pl.when(i == 0)
    def _():
        g8_ref[...] = jnp.zeros_like(g8_ref)

    p = i // _BPG
    rowsel = lax.broadcasted_iota(jnp.int32, (8, _D), 0) == p
    g8_ref[...] += jnp.where(rowsel, jnp.sum(x, axis=0, keepdims=True), 0.0)

    @pl.when(i == _NBLK - 1)
    def _():
        g8 = g8_ref[...]
        sup = jnp.tanh(jnp.dot(g8, w1t_ref[...],
                               preferred_element_type=jnp.float32))
        logits = jnp.dot(sup, w2t_ref[...],
                         preferred_element_type=jnp.float32)
        row = lax.broadcasted_iota(jnp.int32, (8, _NEXP), 0)
        logits = jnp.where(row < _P, logits, -1e30)
        m = jnp.max(logits, axis=0, keepdims=True)
        e = jnp.exp(logits - m)
        att = e / jnp.sum(e, axis=0, keepdims=True)
        o_ref[...] = lax.dot_general(att, g8, (((0,), (0,)), ((), ())),
                                     preferred_element_type=jnp.float32)


_w_spec = pl.BlockSpec((_D, _D), lambda i: (0, 0))
_v_spec = pl.BlockSpec((1, _D), lambda i: (0, 0))
_row3_spec = pl.BlockSpec((1, _ROWB, _D), lambda i: (i // _BPG, i % _BPG, 0))

_mlp_mid = pl.pallas_call(
    _mlp_mid_body,
    grid=(_NBLK,),
    in_specs=[_row3_spec, _row3_spec, _w_spec, _v_spec, _w_spec, _v_spec,
              _v_spec, _v_spec],
    out_specs=_row3_spec,
    out_shape=jax.ShapeDtypeStruct((_P, _N, _D), jnp.float32),
)

_mlp_last = pl.pallas_call(
    _mlp_last_body,
    grid=(_NBLK,),
    in_specs=[_row3_spec, _w_spec, _v_spec, _w_spec, _v_spec, _v_spec,
              _v_spec,
              pl.BlockSpec((_D, _NATT), lambda i: (0, 0)),
              pl.BlockSpec((_NATT, _NEXP), lambda i: (0, 0))],
    out_specs=pl.BlockSpec((_NEXP, _D), lambda i: (0, 0)),
    out_shape=jax.ShapeDtypeStruct((_NEXP, _D), jnp.float32),
    scratch_shapes=[pltpu.VMEM((8, _D), jnp.float32)],
)


def kernel(decomp_x, decomp_edge_index, decomp_edge_attr,
           Wa, ba, Wb, bb, lng, lnb, attw1, attw2):
    del decomp_edge_attr
    x3 = decomp_x

    out8 = None
    for l in range(_L):
        hp = _seg(x3, decomp_edge_index)
        ba_l, bb_l = ba[l].reshape(1, _D), bb[l].reshape(1, _D)
        g_l, b_l = lng[l].reshape(1, _D), lnb[l].reshape(1, _D)
        if l < _L - 1:
            x3 = _mlp_mid(hp, x3, Wa[l], ba_l, Wb[l], bb_l, g_l, b_l)
        else:
            out8 = _mlp_last(hp, Wa[l], ba_l, Wb[l], bb_l, g_l, b_l,
                             attw1.T, attw2.T)

    return out8.reshape(1, _NEXP * _D)


# speedup vs baseline: 1.0912x; 1.0033x over previous
import jax
import jax.numpy as jnp
from jax import lax
from jax.experimental import pallas as pl
from jax.experimental.pallas import tpu as pltpu
from jax.experimental.pallas import tpu_sc as plsc

_P, _N, _E, _D = 2, 10000, 160000, 128
_L = 3
_NEXP, _NATT = 8, 64
_NS = 16
_NC = 2
_CH = 80
_EPT = _E // _NS
_NCHUNK = _EPT // _CH
_G = 4
_IB = 6
_LCM = 12
_NLOOP = _NCHUNK // _LCM
_TAIL = _NCHUNK - _NLOOP * _LCM
_RPT = _N // _NS


def _seg_body(x_hbm, ei_hbm, out_hbm,
              sidx, didx, rows, i0, i1, i2, i3, i4, i5, g0, g1, g2, g3, acc):
    isems = (i0, i1, i2, i3, i4, i5)
    gsems = (g0, g1, g2, g3)
    c = lax.axis_index("c")
    s = lax.axis_index("s")
    ebase = s * _EPT
    r0 = s * _RPT

    def idx_issue(k, slot):
        pltpu.async_copy(ei_hbm.at[c, 0, pl.ds(ebase + k * _CH, _CH)],
                         sidx.at[slot], isems[slot])
        pltpu.async_copy(ei_hbm.at[c, 1, pl.ds(ebase + k * _CH, _CH)],
                         didx.at[slot], isems[slot])

    def gather_issue(slot, rslot):
        pltpu.make_async_copy(ei_hbm.at[c, 0, pl.ds(0, _CH)],
                              sidx.at[slot], isems[slot]).wait()
        pltpu.make_async_copy(ei_hbm.at[c, 0, pl.ds(0, _CH)],
                              didx.at[slot], isems[slot]).wait()
        pltpu.async_copy(x_hbm.at[c].at[sidx.at[slot]], rows.at[rslot],
                         gsems[rslot])

    def turn(i, b):
        ib = b % _IB
        rb = b % _G
        pltpu.make_async_copy(x_hbm.at[c].at[sidx.at[ib]], rows.at[rb],
                              gsems[rb]).wait()
        pltpu.sync_copy(rows.at[rb], acc.at[didx.at[ib]], add=True)

        @pl.when(i + _IB < _NCHUNK)
        def _():
            idx_issue(i + _IB, ib)

        @pl.when(i + _G < _NCHUNK)
        def _():
            gather_issue((b + _G) % _IB, rb)

    for k in range(_IB):
        idx_issue(k, k)
    pltpu.sync_copy(x_hbm.at[c, pl.ds(r0, _RPT)], acc.at[pl.ds(r0, _RPT)])
    plsc.subcore_barrier()
    for g in range(_G):
        gather_issue(g, g)

    def step(j, carry):
        base = j * _LCM
        for b in range(_LCM):
            turn(base + b, b)
        return carry

    lax.fori_loop(0, _NLOOP, step, 0)
    for k in range(_TAIL):
        turn(_NLOOP * _LCM + k, k)

    plsc.subcore_barrier()
    pltpu.sync_copy(acc.at[pl.ds(r0, _RPT)], out_hbm.at[c, pl.ds(r0, _RPT)])


_seg_call = None


def _seg(x3, ei):
    global _seg_call
    if _seg_call is None:
        _seg_call = pl.kernel(
            _seg_body,
            out_type=jax.ShapeDtypeStruct((_P, _N, _D), jnp.float32),
            mesh=plsc.VectorSubcoreMesh(core_axis_name="c",
                                        subcore_axis_name="s",
                                        num_cores=_NC, num_subcores=_NS),
            compiler_params=pltpu.CompilerParams(use_tc_tiling_on_sc=False),
            scratch_types=(
                [pltpu.VMEM((_IB, _CH), jnp.int32),
                 pltpu.VMEM((_IB, _CH), jnp.int32),
                 pltpu.VMEM((_G, _CH, _D), jnp.float32)]
                + [pltpu.SemaphoreType.DMA] * (_IB + _G)
                + [pltpu.VMEM_SHARED((_N, _D), jnp.float32)]
            ),
        )
    return _seg_call(x3, ei)


_ROWB = 10000
_NBLK = (_P * _N) // _ROWB
_BPG = _N // _ROWB


def _ln(t, g, b):
    mu = jnp.mean(t, axis=-1, keepdims=True)
    var = jnp.mean((t - mu) * (t - mu), axis=-1, keepdims=True)
    return (t - mu) * lax.rsqrt(var + 1e-5) * g + b


def _dot_hp(a, w):
    return jnp.dot(a, w, preferred_element_type=jnp.float32)


def _mlp_mid_body(hp_ref, idn_ref, wa_ref, ba_ref, wb_ref, bb_ref,
                  g_ref, b_ref, o_ref):
    h1 = jnp.maximum(_dot_hp(hp_ref[0], wa_ref[...]) + ba_ref[...], 0.0)
    h2 = _dot_hp(h1, wb_ref[...]) + bb_ref[...] + idn_ref[0]
    o_ref[0] = _ln(h2, g_ref[...], b_ref[...])


def _mlp_last_body(hp_ref, wa_ref, ba_ref, wb_ref, bb_ref,
                   g_ref, b_ref, w1t_ref, w2t_ref, o_ref, g8_ref):
    i = pl.program_id(0)
    h1 = jnp.maximum(_dot_hp(hp_ref[0], wa_ref[...]) + ba_ref[...], 0.0)
    h2 = _dot_hp(h1, wb_ref[...]) + bb_ref[...]
    x = _ln(h2, g_ref[...], b_ref[...])

    @pl.when(i == 0)
    def _():
        g8_ref[...] = jnp.zeros_like(g8_ref)

    p = i // _BPG
    rowsel = lax.broadcasted_iota(jnp.int32, (8, _D), 0) == p
    g8_ref[...] += jnp.where(rowsel, jnp.sum(x, axis=0, keepdims=True), 0.0)

    @pl.when(i == _NBLK - 1)
    def _():
        g8 = g8_ref[...]
        sup = jnp.tanh(jnp.dot(g8, w1t_ref[...],
                               preferred_element_type=jnp.float32))
        logits = jnp.dot(sup, w2t_ref[...],
                         preferred_element_type=jnp.float32)
        row = lax.broadcasted_iota(jnp.int32, (8, _NEXP), 0)
        logits = jnp.where(row < _P, logits, -1e30)
        m = jnp.max(logits, axis=0, keepdims=True)
        e = jnp.exp(logits - m)
        att = e / jnp.sum(e, axis=0, keepdims=True)
        o_ref[...] = lax.dot_general(att, g8, (((0,), (0,)), ((), ())),
                                     preferred_element_type=jnp.float32)


_w_spec = pl.BlockSpec((_D, _D), lambda i: (0, 0))
_v_spec = pl.BlockSpec((1, _D), lambda i: (0, 0))
_row3_spec = pl.BlockSpec((1, _ROWB, _D), lambda i: (i // _BPG, i % _BPG, 0))

_mlp_mid = pl.pallas_call(
    _mlp_mid_body,
    grid=(_NBLK,),
    in_specs=[_row3_spec, _row3_spec, _w_spec, _v_spec, _w_spec, _v_spec,
              _v_spec, _v_spec],
    out_specs=_row3_spec,
    out_shape=jax.ShapeDtypeStruct((_P, _N, _D), jnp.float32),
)

_mlp_last = pl.pallas_call(
    _mlp_last_body,
    grid=(_NBLK,),
    in_specs=[_row3_spec, _w_spec, _v_spec, _w_spec, _v_spec, _v_spec,
              _v_spec,
              pl.BlockSpec((_D, _NATT), lambda i: (0, 0)),
              pl.BlockSpec((_NATT, _NEXP), lambda i: (0, 0))],
    out_specs=pl.BlockSpec((_NEXP, _D), lambda i: (0, 0)),
    out_shape=jax.ShapeDtypeStruct((_NEXP, _D), jnp.float32),
    scratch_shapes=[pltpu.VMEM((8, _D), jnp.float32)],
)


def kernel(decomp_x, decomp_edge_index, decomp_edge_attr,
           Wa, ba, Wb, bb, lng, lnb, attw1, attw2):
    del decomp_edge_attr
    x3 = decomp_x

    out8 = None
    for l in range(_L):
        hp = _seg(x3, decomp_edge_index)
        ba_l, bb_l = ba[l].reshape(1, _D), bb[l].reshape(1, _D)
        g_l, b_l = lng[l].reshape(1, _D), lnb[l].reshape(1, _D)
        if l < _L - 1:
            x3 = _mlp_mid(hp, x3, Wa[l], ba_l, Wb[l], bb_l, g_l, b_l)
        else:
            out8 = _mlp_last(hp, Wa[l], ba_l, Wb[l], bb_l, g_l, b_l,
                             attw1.T, attw2.T)

    return out8.reshape(1, _NEXP * _D)
